# baseline (speedup 1.0000x reference)
import functools

import jax
import jax.numpy as jnp
from jax import lax
from jax.experimental import pallas as pl
from jax.experimental.pallas import tpu as pltpu
from jax.experimental.pallas import tpu_sc as plsc

B, C, H, W = 16, 34, 128, 128
K = 32
HW = H * W
L = 16
NC, NS = 2, 16
NW = NC * NS
PAIRS = B * K
PPT = PAIRS // NW
EPT = PPT * C
ROWS = B * C * HW // L

_CHUNKS = [(0, 128), (128, 128), (256, 128), (384, 128), (512, 32)]


@functools.partial(
    pl.kernel,
    out_type=jax.ShapeDtypeStruct((NC, 2 * L), jnp.float32),
    mesh=plsc.VectorSubcoreMesh(
        core_axis_name="c", subcore_axis_name="s", num_cores=NC, num_subcores=NS
    ),
    compiler_params=pltpu.CompilerParams(
        needs_layout_passes=False, use_tc_tiling_on_sc=False
    ),
    scratch_types=[
        pltpu.VMEM((PPT,), jnp.int32),
        pltpu.VMEM((EPT,), jnp.int32),
        pltpu.VMEM((EPT, L), jnp.float32),
        pltpu.VMEM((PPT, C), jnp.float32),
        pltpu.VMEM((PPT, C), jnp.float32),
        pltpu.VMEM((2 * L,), jnp.float32),
        pltpu.VMEM((NS, 2 * L), jnp.float32),
        pltpu.VMEM_SHARED((NS, 2 * L), jnp.float32),
        pltpu.SemaphoreType.DMA,
    ],
)
def _wl1_sc(table, mask3, ind2, targ3, out, ind_v, idx_v, rows_v, mask_v,
            targ_v, parts_v, allp_v, shared, sem):
    cid = lax.axis_index("c")
    sid = lax.axis_index("s")
    wid = cid * NS + sid
    b = wid // (K // PPT)
    k0 = (wid % (K // PPT)) * PPT
    lane = lax.broadcasted_iota(jnp.int32, (L,), 0)

    pltpu.sync_copy(ind2.at[b, pl.ds(k0, PPT)], ind_v)
    iv = ind_v[...]
    rem = jnp.bitwise_and(iv, L - 1)

    row0 = lax.shift_right_logical(iv + b * C * HW, 4)

    def _build(c, carry):
        off = pl.multiple_of(c * L, L)
        idx_v[pl.ds(off, L)] = row0 + c * (HW // L)
        return carry

    lax.fori_loop(0, C, _build, 0, unroll=2)

    copies = [
        pltpu.async_copy(table.at[idx_v.at[pl.ds(off, n)]],
                         rows_v.at[pl.ds(off, n)], sem)
        for off, n in _CHUNKS
    ]
    pltpu.sync_copy(mask3.at[b, pl.ds(k0, PPT)], mask_v)
    pltpu.sync_copy(targ3.at[b, pl.ds(k0, PPT)], targ_v)
    for cp in copies:
        cp.wait()

    def _accum(c, carry):
        acc, msum = carry
        cs = jnp.full((L,), 0, jnp.int32) + c
        p = plsc.load_gather(rows_v, [c * L + lane, rem])
        m = plsc.load_gather(mask_v, [lane, cs])
        t = plsc.load_gather(targ_v, [lane, cs])
        return acc + jnp.abs(p * m - t * m), msum + m

    acc, msum = lax.fori_loop(
        0, C, _accum,
        (jnp.zeros((L,), jnp.float32), jnp.zeros((L,), jnp.float32)),
        unroll=2,
    )

    parts_v[pl.ds(0, L)] = acc
    parts_v[pl.ds(L, L)] = msum
    pltpu.sync_copy(parts_v, shared.at[sid])
    plsc.subcore_barrier()

    @pl.when(sid == 0)
    def _():
        pltpu.sync_copy(shared, allp_v)

        def _red(r, carry):
            a, m2 = carry
            return (a + allp_v[r, pl.ds(0, L)],
                    m2 + allp_v[r, pl.ds(L, L)])

        a, m2 = lax.fori_loop(
            0, NS, _red,
            (jnp.zeros((L,), jnp.float32), jnp.zeros((L,), jnp.float32)),
        )
        parts_v[pl.ds(0, L)] = a
        parts_v[pl.ds(L, L)] = m2
        pltpu.sync_copy(parts_v, out.at[cid])


def kernel(output, mask, ind, target):
    table = output.reshape(ROWS, L)
    parts = _wl1_sc(table, mask.astype(jnp.float32), ind.astype(jnp.int32),
                    target.astype(jnp.float32))
    l1 = jnp.sum(parts[:, :L])
    msum = jnp.sum(parts[:, L:])
    return l1 / (msum + 1e-4)


# speedup vs baseline: 1.0012x; 1.0012x over previous
import functools

import jax
import jax.numpy as jnp
from jax import lax
from jax.experimental import pallas as pl
from jax.experimental.pallas import tpu as pltpu
from jax.experimental.pallas import tpu_sc as plsc

B, C, H, W = 16, 34, 128, 128
K = 32
HW = H * W
L = 16
NC, NS = 2, 16
NW = NC * NS
PAIRS = B * K
PPT = PAIRS // NW
EPT = PPT * C
ROWS = B * C * HW // L

_CHUNKS = [(0, 128), (128, 128), (256, 128), (384, 128), (512, 32)]


@functools.partial(
    pl.kernel,
    out_type=jax.ShapeDtypeStruct((NC, 2 * L), jnp.float32),
    mesh=plsc.VectorSubcoreMesh(
        core_axis_name="c", subcore_axis_name="s", num_cores=NC, num_subcores=NS
    ),
    compiler_params=pltpu.CompilerParams(
        needs_layout_passes=False, use_tc_tiling_on_sc=False,
        skip_device_barrier=True,
    ),
    scratch_types=[
        pltpu.VMEM((PPT,), jnp.int32),
        pltpu.VMEM((EPT,), jnp.int32),
        pltpu.VMEM((EPT, L), jnp.float32),
        pltpu.VMEM((PPT, C), jnp.float32),
        pltpu.VMEM((PPT, C), jnp.float32),
        pltpu.VMEM((2 * L,), jnp.float32),
        pltpu.VMEM((NS, 2 * L), jnp.float32),
        pltpu.VMEM_SHARED((NS, 2 * L), jnp.float32),
        pltpu.SemaphoreType.DMA,
    ],
)
def _wl1_sc(table, mask3, ind2, targ3, out, ind_v, idx_v, rows_v, mask_v,
            targ_v, parts_v, allp_v, shared, sem):
    cid = lax.axis_index("c")
    sid = lax.axis_index("s")
    wid = cid * NS + sid
    b = wid // (K // PPT)
    k0 = (wid % (K // PPT)) * PPT
    lane = lax.broadcasted_iota(jnp.int32, (L,), 0)

    pltpu.sync_copy(ind2.at[b, pl.ds(k0, PPT)], ind_v)
    iv = ind_v[...]
    rem = jnp.bitwise_and(iv, L - 1)

    row0 = lax.shift_right_logical(iv + b * C * HW, 4)

    def _build(c, carry):
        off = pl.multiple_of(c * L, L)
        idx_v[pl.ds(off, L)] = row0 + c * (HW // L)
        return carry

    lax.fori_loop(0, C, _build, 0, unroll=2)

    copies = [
        pltpu.async_copy(table.at[idx_v.at[pl.ds(off, n)]],
                         rows_v.at[pl.ds(off, n)], sem)
        for off, n in _CHUNKS
    ]
    pltpu.sync_copy(mask3.at[b, pl.ds(k0, PPT)], mask_v)
    pltpu.sync_copy(targ3.at[b, pl.ds(k0, PPT)], targ_v)
    for cp in copies:
        cp.wait()

    def _accum(c, carry):
        acc, msum = carry
        cs = jnp.full((L,), 0, jnp.int32) + c
        p = plsc.load_gather(rows_v, [c * L + lane, rem])
        m = plsc.load_gather(mask_v, [lane, cs])
        t = plsc.load_gather(targ_v, [lane, cs])
        return acc + jnp.abs(p * m - t * m), msum + m

    acc, msum = lax.fori_loop(
        0, C, _accum,
        (jnp.zeros((L,), jnp.float32), jnp.zeros((L,), jnp.float32)),
        unroll=2,
    )

    parts_v[pl.ds(0, L)] = acc
    parts_v[pl.ds(L, L)] = msum
    pltpu.sync_copy(parts_v, shared.at[sid])
    plsc.subcore_barrier()

    @pl.when(sid == 0)
    def _():
        pltpu.sync_copy(shared, allp_v)

        def _red(r, carry):
            a, m2 = carry
            return (a + allp_v[r, pl.ds(0, L)],
                    m2 + allp_v[r, pl.ds(L, L)])

        a, m2 = lax.fori_loop(
            0, NS, _red,
            (jnp.zeros((L,), jnp.float32), jnp.zeros((L,), jnp.float32)),
        )
        parts_v[pl.ds(0, L)] = a
        parts_v[pl.ds(L, L)] = m2
        pltpu.sync_copy(parts_v, out.at[cid])


def kernel(output, mask, ind, target):
    table = output.reshape(ROWS, L)
    parts = _wl1_sc(table, mask.astype(jnp.float32), ind.astype(jnp.int32),
                    target.astype(jnp.float32))
    l1 = jnp.sum(parts[:, :L])
    msum = jnp.sum(parts[:, L:])
    return l1 / (msum + 1e-4)


# speedup vs baseline: 1.1354x; 1.1340x over previous
import functools

import jax
import jax.numpy as jnp
from jax import lax
from jax.experimental import pallas as pl
from jax.experimental.pallas import tpu as pltpu
from jax.experimental.pallas import tpu_sc as plsc

B, C, H, W = 16, 34, 128, 128
K = 32
HW = H * W
L = 16
NC, NS = 2, 16
NW = NC * NS
PAIRS = B * K
PPT = PAIRS // NW
EPT = PPT * C
ROWS = B * C * HW // L
CP = 128

_CHUNKS = [(0, 128), (128, 128), (256, 128), (384, 128), (512, 32)]


@functools.partial(
    pl.kernel,
    out_type=jax.ShapeDtypeStruct((PAIRS, CP), jnp.float32),
    mesh=plsc.VectorSubcoreMesh(
        core_axis_name="c", subcore_axis_name="s", num_cores=NC, num_subcores=NS
    ),
    compiler_params=pltpu.CompilerParams(
        needs_layout_passes=False, use_tc_tiling_on_sc=False,
        skip_device_barrier=True,
    ),
    scratch_types=[
        pltpu.VMEM((PPT,), jnp.int32),
        pltpu.VMEM((EPT,), jnp.int32),
        pltpu.VMEM((EPT, L), jnp.float32),
        pltpu.VMEM((PPT, CP), jnp.float32),
        pltpu.SemaphoreType.DMA,
    ],
)
def _gather_sc(table, indf, out, ind_v, idx_v, rows_v, pred_v, sem):
    cid = lax.axis_index("c")
    sid = lax.axis_index("s")
    wid = cid * NS + sid
    b = wid // (K // PPT)
    lane = lax.broadcasted_iota(jnp.int32, (L,), 0)

    pltpu.sync_copy(indf.at[pl.ds(wid * PPT, PPT)], ind_v)
    iv = ind_v[...]

    row0 = lax.shift_right_logical(iv + b * C * HW, 4)

    def _build(c, carry):
        off = pl.multiple_of(c * L, L)
        idx_v[pl.ds(off, L)] = row0 + c * (HW // L)
        return carry

    lax.fori_loop(0, C, _build, 0, unroll=2)

    for cp in [
        pltpu.async_copy(table.at[idx_v.at[pl.ds(off, n)]],
                         rows_v.at[pl.ds(off, n)], sem)
        for off, n in _CHUNKS
    ]:
        cp.wait()

    def _row(j, carry):
        js = jnp.full((L,), 0, jnp.int32) + j
        rem_j = jnp.bitwise_and(plsc.load_gather(ind_v, [js]), L - 1)
        for chunk in range(3):
            cs = jnp.minimum(lane + chunk * L, C - 1)
            vals = plsc.load_gather(rows_v, [cs * L + js, rem_j])
            pred_v[j, pl.ds(chunk * L, L)] = vals
        return carry

    lax.fori_loop(0, PPT, _row, 0, unroll=2)

    pltpu.sync_copy(pred_v, out.at[pl.ds(wid * PPT, PPT)])


def _reduce_tc(pred_ref, mask_ref, targ_ref, out_ref):
    p = pred_ref[...][:, :C].reshape(B, K, C)
    m = mask_ref[...]
    t = targ_ref[...]
    l1 = jnp.sum(jnp.abs(p * m - t * m))
    ms = jnp.sum(m)
    out_ref[0, 0] = l1 / (ms + 1e-4)


def kernel(output, mask, ind, target):
    table = output.reshape(ROWS, L)
    indf = ind.reshape(-1).astype(jnp.int32)
    pred = _gather_sc(table, indf)
    loss = pl.pallas_call(
        _reduce_tc,
        out_shape=jax.ShapeDtypeStruct((1, 1), jnp.float32),
        out_specs=pl.BlockSpec(memory_space=pltpu.SMEM),
    )(pred, mask.astype(jnp.float32), target.astype(jnp.float32))
    return loss[0, 0]


# speedup vs baseline: 1.1456x; 1.0089x over previous
import functools

import jax
import jax.numpy as jnp
from jax import lax
from jax.experimental import pallas as pl
from jax.experimental.pallas import tpu as pltpu
from jax.experimental.pallas import tpu_sc as plsc

B, C, H, W = 16, 34, 128, 128
K = 32
HW = H * W
L = 16
NC, NS = 2, 16
NW = NC * NS
PAIRS = B * K
PPT = PAIRS // NW
EPT = PPT * C
ROWS = B * C * HW // L
CP = 128

_CHUNKS = [(0, 128), (128, 128), (256, 128), (384, 128), (512, 32)]


@functools.partial(
    pl.kernel,
    out_type=jax.ShapeDtypeStruct((PAIRS, CP), jnp.float32),
    mesh=plsc.VectorSubcoreMesh(
        core_axis_name="c", subcore_axis_name="s", num_cores=NC, num_subcores=NS
    ),
    compiler_params=pltpu.CompilerParams(
        needs_layout_passes=False, use_tc_tiling_on_sc=False,
        skip_device_barrier=True,
    ),
    scratch_types=[
        pltpu.VMEM((PPT,), jnp.int32),
        pltpu.VMEM((EPT,), jnp.int32),
        pltpu.VMEM((EPT, L), jnp.float32),
        pltpu.VMEM((PPT, CP), jnp.float32),
        pltpu.SemaphoreType.DMA,
    ],
)
def _gather_sc(table, indf, out, ind_v, idx_v, rows_v, pred_v, sem):
    cid = lax.axis_index("c")
    sid = lax.axis_index("s")
    wid = cid * NS + sid
    b = wid // (K // PPT)
    lane = lax.broadcasted_iota(jnp.int32, (L,), 0)

    pltpu.sync_copy(indf.at[pl.ds(wid * PPT, PPT)], ind_v)
    iv = ind_v[...]

    row0 = lax.shift_right_logical(iv + b * C * HW, 4)

    def _build(c, carry):
        off = pl.multiple_of(c * L, L)
        idx_v[pl.ds(off, L)] = row0 + c * (HW // L)
        return carry

    lax.fori_loop(0, C, _build, 0, unroll=4)

    for cp in [
        pltpu.async_copy(table.at[idx_v.at[pl.ds(off, n)]],
                         rows_v.at[pl.ds(off, n)], sem)
        for off, n in _CHUNKS
    ]:
        cp.wait()

    def _row(j, carry):
        js = jnp.full((L,), 0, jnp.int32) + j
        rem_j = jnp.bitwise_and(plsc.load_gather(ind_v, [js]), L - 1)
        for chunk in range(3):
            cs = jnp.minimum(lane + chunk * L, C - 1)
            vals = plsc.load_gather(rows_v, [cs * L + js, rem_j])
            pred_v[j, pl.ds(chunk * L, L)] = vals
        return carry

    lax.fori_loop(0, PPT, _row, 0, unroll=4)

    pltpu.sync_copy(pred_v, out.at[pl.ds(wid * PPT, PPT)])


def _reduce_tc(pred_ref, mask_ref, targ_ref, out_ref):
    p = pred_ref[...][:, :C].reshape(B, K, C)
    m = mask_ref[...]
    t = targ_ref[...]
    l1 = jnp.sum(jnp.abs(p * m - t * m))
    ms = jnp.sum(m)
    out_ref[0, 0] = l1 / (ms + 1e-4)


def kernel(output, mask, ind, target):
    table = output.reshape(ROWS, L)
    indf = ind.reshape(-1).astype(jnp.int32)
    pred = _gather_sc(table, indf)
    loss = pl.pallas_call(
        _reduce_tc,
        out_shape=jax.ShapeDtypeStruct((1, 1), jnp.float32),
        out_specs=pl.BlockSpec(memory_space=pltpu.SMEM),
    )(pred, mask.astype(jnp.float32), target.astype(jnp.float32))
    return loss[0, 0]
